# baseline (speedup 1.0000x reference)
import functools

import jax
import jax.numpy as jnp
import numpy as np
from jax import lax
from jax.experimental import pallas as pl
from jax.experimental.pallas import tpu as pltpu
from jax.experimental.pallas import tpu_sc as plsc

NODE = ["object", "ssBox", "place_frame", "ssCylinder", "pick", "place"]
INDIMS = [8, 8, 8, 7, 4, 4]
ET = [("object", "ssBox"), ("ssBox", "object"), ("place_frame", "ssBox"),
      ("ssBox", "place_frame"), ("place_frame", "object"), ("object", "place_frame"),
      ("pick", "place"), ("place", "pick"), ("object", "object"),
      ("ssBox", "ssBox"), ("place_frame", "place_frame"), ("ssCylinder", "ssCylinder"),
      ("object", "pick"), ("pick", "object"), ("place_frame", "pick"),
      ("pick", "place_frame"), ("ssCylinder", "pick"), ("pick", "ssCylinder"),
      ("object", "place"), ("place", "object"), ("ssCylinder", "place"),
      ("place", "ssCylinder"), ("place_frame", "place"), ("place", "place_frame")]
TIDX = {t: i for i, t in enumerate(NODE)}
NT = 6
NE = 24
N = 50000
E = 50000
H = 64
L = 3

ORDER = sorted(range(NE), key=lambda i: TIDX[ET[i][1]])
SRC_P = [TIDX[ET[i][0]] for i in ORDER]
DST_P = [TIDX[ET[i][1]] for i in ORDER]
FIRST_MASK = 0
LAST_MASK = 0
for j in range(NE):
    if j == 0 or DST_P[j] != DST_P[j - 1]:
        FIRST_MASK |= 1 << j
    if j == NE - 1 or DST_P[j] != DST_P[j + 1]:
        LAST_MASK |= 1 << j

NTILE = 16
CH = 128
NCHUNK = 25
EPT = CH * NCHUNK
E_PAD = EPT * NTILE
SEG = 3128
ACC_ROWS = SEG * NTILE
ZR = 128
R = 2000
NRB = N // R
GROUPS = [[j for j in range(NE) if DST_P[j] == t] for t in range(NT)]
HH = H // 2


def _embed_body(x_ref, w_ref, b_ref, o_ref):
    o_ref[0, 0] = (
        jnp.dot(x_ref[0], w_ref[0, 0], preferred_element_type=jnp.float32, precision=jax.lax.Precision.HIGHEST)
        + b_ref[0, 0]
    )


def _embed(xs, w3, b3):
    return pl.pallas_call(
        _embed_body,
        grid=(2, NT, NRB),
        in_specs=[
            pl.BlockSpec((1, R, 8), lambda c, t, r: (t, r, 0)),
            pl.BlockSpec((1, 1, 8, HH), lambda c, t, r: (c, t, 0, 0)),
            pl.BlockSpec((1, 1, 1, HH), lambda c, t, r: (c, t, 0, 0)),
        ],
        out_specs=pl.BlockSpec((1, 1, R, HH), lambda c, t, r: (c, t, r, 0)),
        out_shape=jax.ShapeDtypeStruct((2, NT, N, HH), jnp.float32),
    )(xs, w3, b3)


def _counts(e1):
    mesh = plsc.VectorSubcoreMesh(core_axis_name="c", subcore_axis_name="s")
    per_core = NE // 2

    @functools.partial(
        pl.kernel,
        out_type=jax.ShapeDtypeStruct((NE, N), jnp.float32),
        mesh=mesh,
        compiler_params=pltpu.CompilerParams(use_tc_tiling_on_sc=False),
        scratch_types=[
            pltpu.VMEM_SHARED((per_core, ACC_ROWS), jnp.float32),
            pltpu.VMEM((NCHUNK, CH), jnp.int32),
            pltpu.VMEM((CH,), jnp.float32),
            pltpu.VMEM((SEG,), jnp.float32),
        ],
    )
    def k(e1_hbm, cnt_hbm, acc, idxb, ones, zbuf):
        c = lax.axis_index("c")
        s = lax.axis_index("s")

        @pl.loop(0, CH, step=16)
        def _(j):
            ones[pl.ds(j, 16)] = jnp.full((16,), 1.0, jnp.float32)

        @pl.loop(0, SEG - 8, step=16)
        def _(j):
            zbuf[pl.ds(j, 16)] = jnp.zeros((16,), jnp.float32)
        zbuf[pl.ds(SEG - 16, 16)] = jnp.zeros((16,), jnp.float32)

        for ii in range(per_core):
            pltpu.sync_copy(zbuf, acc.at[ii, pl.ds(s * SEG, SEG)])
        plsc.subcore_barrier()
        for ii in range(per_core):
            pltpu.sync_copy(e1_hbm.at[c * per_core + ii, s], idxb)

            @pl.loop(0, NCHUNK)
            def _(j, _ii=ii):
                pltpu.sync_copy(ones, acc.at[_ii].at[idxb.at[j]], add=True)

        plsc.subcore_barrier()
        for ii in range(per_core):
            @pl.when(s < NTILE - 1)
            def _(_ii=ii):
                pltpu.sync_copy(
                    acc.at[_ii, pl.ds(s * SEG, SEG)],
                    cnt_hbm.at[c * per_core + _ii, pl.ds(s * SEG, SEG)])

            @pl.when(s == NTILE - 1)
            def _(_ii=ii):
                last = N - (NTILE - 1) * SEG
                pltpu.sync_copy(
                    acc.at[_ii, pl.ds((NTILE - 1) * SEG, last)],
                    cnt_hbm.at[c * per_core + _ii, pl.ds((NTILE - 1) * SEG, last)])

    return k(e1)


def _messages(h2, e0, e1):
    mesh = plsc.VectorSubcoreMesh(core_axis_name="c", subcore_axis_name="s")

    @functools.partial(
        pl.kernel,
        out_type=jax.ShapeDtypeStruct((2 * NE, N, HH), jnp.float32),
        mesh=mesh,
        compiler_params=pltpu.CompilerParams(use_tc_tiling_on_sc=False),
        scratch_types=[
            pltpu.VMEM_SHARED((ACC_ROWS, HH), jnp.float32),
            pltpu.VMEM((NCHUNK, CH), jnp.int32),
            pltpu.VMEM((NCHUNK, CH), jnp.int32),
            pltpu.VMEM((CH, HH), jnp.float32),
            pltpu.VMEM((ZR, HH), jnp.float32),
        ],
    )
    def k(h_hbm, e0_hbm, e1_hbm, msg_hbm, acc, idx0, idx1, rows, zrows):
        c = lax.axis_index("c")
        s = lax.axis_index("s")

        @pl.loop(0, ZR)
        def _(j):
            zrows[j, pl.ds(0, 16)] = jnp.zeros((16,), jnp.float32)
            zrows[j, pl.ds(16, 16)] = jnp.zeros((16,), jnp.float32)

        for i in range(NE):
            @pl.loop(0, 24)
            def _(z):
                pltpu.sync_copy(zrows, acc.at[pl.ds(s * SEG + z * ZR, ZR)])
            pltpu.sync_copy(zrows.at[pl.ds(0, SEG - 24 * ZR)],
                            acc.at[pl.ds(s * SEG + 24 * ZR, SEG - 24 * ZR)])
            plsc.subcore_barrier()
            pltpu.sync_copy(e0_hbm.at[i, s], idx0)
            pltpu.sync_copy(e1_hbm.at[i, s], idx1)
            src = SRC_P[i]

            @pl.loop(0, NCHUNK)
            def _(j, _src=src):
                pltpu.sync_copy(h_hbm.at[c * NT + _src].at[idx0.at[j]], rows)
                pltpu.sync_copy(rows, acc.at[idx1.at[j]], add=True)

            plsc.subcore_barrier()

            @pl.when(s < NTILE - 1)
            def _(_i=i):
                pltpu.sync_copy(
                    acc.at[pl.ds(s * SEG, SEG)],
                    msg_hbm.at[c * NE + _i, pl.ds(s * SEG, SEG)])

            @pl.when(s == NTILE - 1)
            def _(_i=i):
                last = N - (NTILE - 1) * SEG
                pltpu.sync_copy(
                    acc.at[pl.ds((NTILE - 1) * SEG, last)],
                    msg_hbm.at[c * NE + _i, pl.ds((NTILE - 1) * SEG, last)])

    return k(h2, e0, e1)


def _make_combine_body(k):
    def body(*refs):
        msg_refs = refs[0:k]
        cnt_refs = refs[k:2 * k]
        h_ref = refs[2 * k]
        w_ref = refs[2 * k + 1]
        b_ref = refs[2 * k + 2]
        o_ref = refs[2 * k + 3]
        parts = []
        for j in range(k):
            m = jnp.concatenate([msg_refs[j][0, 0], msg_refs[j][1, 0]], axis=1)
            cnt = cnt_refs[j][0]
            parts.append(m * (1.0 / jnp.maximum(cnt, 1.0)))
        parts.append(jnp.concatenate([h_ref[0, 0], h_ref[1, 0]], axis=1))
        x = jnp.concatenate(parts, axis=1)
        v = (jnp.dot(x, w_ref[...], preferred_element_type=jnp.float32, precision=jax.lax.Precision.HIGHEST)
             + b_ref[...])
        v = jnp.maximum(v, 0.0)
        o_ref[0] = v[:, :HH]
        o_ref[1] = v[:, HH:]
    return body


def _combine_t(t, msg, cnt3, h, wcat, bias):
    grp = GROUPS[t]
    k = len(grp)
    in_specs = []
    args = []
    for j in grp:
        in_specs.append(
            pl.BlockSpec((2, 1, R, HH), lambda r, _j=j: (0, _j, r, 0)))
        args.append(msg)
    for j in grp:
        in_specs.append(pl.BlockSpec((1, R, 1), lambda r, _j=j: (_j, r, 0)))
        args.append(cnt3)
    in_specs.append(pl.BlockSpec((2, 1, R, HH), lambda r: (0, t, r, 0)))
    args.append(h)
    in_specs.append(pl.BlockSpec(((k + 1) * H, H), lambda r: (0, 0)))
    args.append(wcat)
    in_specs.append(pl.BlockSpec((1, H), lambda r: (0, 0)))
    args.append(bias)
    return pl.pallas_call(
        _make_combine_body(k),
        grid=(NRB,),
        in_specs=in_specs,
        out_specs=pl.BlockSpec((2, R, HH), lambda r: (0, r, 0)),
        out_shape=jax.ShapeDtypeStruct((2, N, HH), jnp.float32),
    )(*args)


def _readout_body(h4_ref, h5_ref, w1_ref, b1_ref, w2_ref, b2_ref, o_ref, acc):
    r = pl.program_id(0)

    @pl.when(r == 0)
    def _():
        acc[...] = jnp.zeros_like(acc)

    blk = (jnp.concatenate([h4_ref[0, 0], h4_ref[1, 0]], axis=1)
           + jnp.concatenate([h5_ref[0, 0], h5_ref[1, 0]], axis=1))
    acc[0, :H] += jnp.sum(blk, axis=0)

    @pl.when(r == NRB - 1)
    def _():
        g = jnp.maximum(acc[0, :H] * (1.0 / N), 0.0).reshape(1, H)
        z = jnp.maximum(
            jnp.dot(g, w1_ref[...], preferred_element_type=jnp.float32, precision=jax.lax.Precision.HIGHEST)
            + b1_ref[...], 0.0)
        o_ref[...] = (jnp.dot(z, w2_ref[...], preferred_element_type=jnp.float32, precision=jax.lax.Precision.HIGHEST)
                      + b2_ref[...])


def _readout(h, w1, b1, w2, b2):
    return pl.pallas_call(
        _readout_body,
        grid=(NRB,),
        in_specs=[
            pl.BlockSpec((2, 1, R, HH), lambda r: (0, 4, r, 0)),
            pl.BlockSpec((2, 1, R, HH), lambda r: (0, 5, r, 0)),
            pl.BlockSpec((H, H // 4), lambda r: (0, 0)),
            pl.BlockSpec((1, H // 4), lambda r: (0, 0)),
            pl.BlockSpec((H // 4, 1), lambda r: (0, 0)),
            pl.BlockSpec((1, 1), lambda r: (0, 0)),
        ],
        out_specs=pl.BlockSpec((1, 1), lambda r: (0, 0)),
        out_shape=jax.ShapeDtypeStruct((1, 1), jnp.float32),
        scratch_shapes=[pltpu.VMEM((8, 128), jnp.float32)],
    )(h, h, w1, b1, w2, b2)


def kernel(x_object, W_emb_object, b_emb_object,
           x_ssBox, W_emb_ssBox, b_emb_ssBox,
           x_place_frame, W_emb_place_frame, b_emb_place_frame,
           x_ssCylinder, W_emb_ssCylinder, b_emb_ssCylinder,
           x_pick, W_emb_pick, b_emb_pick,
           x_place, W_emb_place, b_emb_place,
           edge_index, W_l, b_l, W_r, b_r,
           W_out1, b_out1, W_out2, b_out2):
    xs_raw = [x_object, x_ssBox, x_place_frame, x_ssCylinder, x_pick, x_place]
    ws_raw = [W_emb_object, W_emb_ssBox, W_emb_place_frame, W_emb_ssCylinder,
              W_emb_pick, W_emb_place]
    bs_raw = [b_emb_object, b_emb_ssBox, b_emb_place_frame, b_emb_ssCylinder,
              b_emb_pick, b_emb_place]

    xs = jnp.stack([jnp.pad(x, ((0, 0), (0, 8 - d)))
                    for x, d in zip(xs_raw, INDIMS)])
    wemb = jnp.stack([jnp.pad(w, ((0, 8 - d), (0, 0)))
                      for w, d in zip(ws_raw, INDIMS)])
    w3 = wemb.reshape(NT, 8, 2, HH).transpose(2, 0, 1, 3)
    b3 = jnp.stack(bs_raw).reshape(NT, 1, 2, HH).transpose(2, 0, 1, 3)

    order = jnp.array(ORDER, jnp.int32)
    ei = jnp.take(edge_index.astype(jnp.int32), order, axis=0)
    pad0 = jnp.broadcast_to((jnp.arange(E_PAD - E, dtype=jnp.int32) * 97) % N,
                            (NE, E_PAD - E))
    pad1 = jnp.broadcast_to(N + (jnp.arange(E_PAD - E, dtype=jnp.int32) % 8),
                            (NE, E_PAD - E))
    e0 = jnp.concatenate([ei[:, 0, :], pad0], axis=1).reshape(NE, NTILE, NCHUNK, CH)
    e1 = jnp.concatenate([ei[:, 1, :], pad1], axis=1).reshape(NE, NTILE, NCHUNK, CH)

    onehot = np.zeros((NT, NE), np.float32)
    for i_orig, (s_t, d_t) in enumerate(ET):
        onehot[TIDX[d_t], i_orig] = 1.0
    oh = jnp.asarray(onehot)
    wr_comb = jnp.einsum("ti,lihk->lthk", oh, W_r)
    bias_comb = jnp.einsum("ti,lih->lth", oh, b_l + b_r).reshape(L, NT, 1, H)
    wl_p = jnp.take(W_l, order, axis=1)

    cnt = _counts(e1)
    cnt3 = cnt.reshape(NE, N, 1)

    wcats = [[jnp.concatenate([wl_p[l, j] for j in GROUPS[t]]
                              + [wr_comb[l, t]], axis=0)
              for t in range(NT)] for l in range(L)]
    biases = [[bias_comb[l, t] for t in range(NT)] for l in range(L)]

    h = _embed(xs, w3, b3)
    for l in range(L):
        msg = _messages(h.reshape(2 * NT, N, HH), e0, e1)
        msg4 = msg.reshape(2, NE, N, HH)
        h = jnp.stack([_combine_t(t, msg4, cnt3, h, wcats[l][t], biases[l][t])
                       for t in range(NT)], axis=1)
    out = _readout(h, W_out1, b_out1.reshape(1, H // 4),
                   W_out2, b_out2.reshape(1, 1))
    return out.reshape(1)


# speedup vs baseline: 1.0838x; 1.0838x over previous
import functools

import jax
import jax.numpy as jnp
import numpy as np
from jax import lax
from jax.experimental import pallas as pl
from jax.experimental.pallas import tpu as pltpu
from jax.experimental.pallas import tpu_sc as plsc

NODE = ["object", "ssBox", "place_frame", "ssCylinder", "pick", "place"]
INDIMS = [8, 8, 8, 7, 4, 4]
ET = [("object", "ssBox"), ("ssBox", "object"), ("place_frame", "ssBox"),
      ("ssBox", "place_frame"), ("place_frame", "object"), ("object", "place_frame"),
      ("pick", "place"), ("place", "pick"), ("object", "object"),
      ("ssBox", "ssBox"), ("place_frame", "place_frame"), ("ssCylinder", "ssCylinder"),
      ("object", "pick"), ("pick", "object"), ("place_frame", "pick"),
      ("pick", "place_frame"), ("ssCylinder", "pick"), ("pick", "ssCylinder"),
      ("object", "place"), ("place", "object"), ("ssCylinder", "place"),
      ("place", "ssCylinder"), ("place_frame", "place"), ("place", "place_frame")]
TIDX = {t: i for i, t in enumerate(NODE)}
NT = 6
NE = 24
N = 50000
E = 50000
H = 64
L = 3

ORDER = sorted(range(NE), key=lambda i: TIDX[ET[i][1]])
SRC_P = [TIDX[ET[i][0]] for i in ORDER]
DST_P = [TIDX[ET[i][1]] for i in ORDER]
FIRST_MASK = 0
LAST_MASK = 0
for j in range(NE):
    if j == 0 or DST_P[j] != DST_P[j - 1]:
        FIRST_MASK |= 1 << j
    if j == NE - 1 or DST_P[j] != DST_P[j + 1]:
        LAST_MASK |= 1 << j

NTILE = 16
CH = 128
NCHUNK = 25
EPT = CH * NCHUNK
E_PAD = EPT * NTILE
SEG = 3200
ACC_ROWS = SEG * NTILE
ZR = 128
R = 2000
NRB = N // R
GROUPS = [[j for j in range(NE) if DST_P[j] == t] for t in range(NT)]
HH = H // 2


def _embed_body(x_ref, w_ref, b_ref, o_ref):
    o_ref[0, 0] = (
        jnp.dot(x_ref[0], w_ref[0, 0], preferred_element_type=jnp.float32, precision=jax.lax.Precision.HIGHEST)
        + b_ref[0, 0]
    )


def _embed(xs, w3, b3):
    return pl.pallas_call(
        _embed_body,
        grid=(2, NT, NRB),
        in_specs=[
            pl.BlockSpec((1, R, 8), lambda c, t, r: (t, r, 0)),
            pl.BlockSpec((1, 1, 8, HH), lambda c, t, r: (c, t, 0, 0)),
            pl.BlockSpec((1, 1, 1, HH), lambda c, t, r: (c, t, 0, 0)),
        ],
        out_specs=pl.BlockSpec((1, 1, R, HH), lambda c, t, r: (c, t, r, 0)),
        out_shape=jax.ShapeDtypeStruct((2, NT, N, HH), jnp.float32),
    )(xs, w3, b3)


def _counts(e1):
    mesh = plsc.VectorSubcoreMesh(core_axis_name="c", subcore_axis_name="s")
    per_core = NE // 2

    @functools.partial(
        pl.kernel,
        out_type=jax.ShapeDtypeStruct((NE, N), jnp.float32),
        mesh=mesh,
        compiler_params=pltpu.CompilerParams(use_tc_tiling_on_sc=False),
        scratch_types=[
            pltpu.VMEM_SHARED((per_core, ACC_ROWS), jnp.float32),
            pltpu.VMEM((NCHUNK, CH), jnp.int32),
            pltpu.VMEM((CH,), jnp.float32),
            pltpu.VMEM((SEG,), jnp.float32),
        ],
    )
    def k(e1_hbm, cnt_hbm, acc, idxb, ones, zbuf):
        c = lax.axis_index("c")
        s = lax.axis_index("s")

        @pl.loop(0, CH, step=16)
        def _(j):
            ones[pl.ds(j, 16)] = jnp.full((16,), 1.0, jnp.float32)

        @pl.loop(0, SEG, step=16)
        def _(j):
            zbuf[pl.ds(j, 16)] = jnp.zeros((16,), jnp.float32)

        for ii in range(per_core):
            pltpu.sync_copy(zbuf, acc.at[ii, pl.ds(s * SEG, SEG)])
        plsc.subcore_barrier()
        for ii in range(per_core):
            pltpu.sync_copy(e1_hbm.at[c * per_core + ii, s], idxb)

            @pl.loop(0, NCHUNK)
            def _(j, _ii=ii):
                pltpu.sync_copy(ones, acc.at[_ii].at[idxb.at[j]], add=True)

        plsc.subcore_barrier()
        for ii in range(per_core):
            @pl.when(s < NTILE - 1)
            def _(_ii=ii):
                pltpu.sync_copy(
                    acc.at[_ii, pl.ds(s * SEG, SEG)],
                    cnt_hbm.at[c * per_core + _ii, pl.ds(s * SEG, SEG)])

            @pl.when(s == NTILE - 1)
            def _(_ii=ii):
                last = N - (NTILE - 1) * SEG
                pltpu.sync_copy(
                    acc.at[_ii, pl.ds((NTILE - 1) * SEG, last)],
                    cnt_hbm.at[c * per_core + _ii, pl.ds((NTILE - 1) * SEG, last)])

    return k(e1)


def _messages(h2, e01):
    mesh = plsc.VectorSubcoreMesh(core_axis_name="c", subcore_axis_name="s")

    @functools.partial(
        pl.kernel,
        out_type=jax.ShapeDtypeStruct((2 * NE, N, HH), jnp.float32),
        mesh=mesh,
        compiler_params=pltpu.CompilerParams(use_tc_tiling_on_sc=False),
        scratch_types=[
            pltpu.VMEM_SHARED((ACC_ROWS, HH), jnp.float32),
            pltpu.VMEM((2, NCHUNK, CH), jnp.int32),
            pltpu.VMEM((4, CH, HH), jnp.float32),
            pltpu.VMEM((ZR, HH), jnp.float32),
            pltpu.SemaphoreType.DMA((4,)),
            pltpu.SemaphoreType.DMA((4,)),
        ],
    )
    def k(h_hbm, e01_hbm, msg_hbm, acc, idx01, rows4, zrows, gsem, ssem):
        idx0 = idx01.at[0]
        idx1 = idx01.at[1]
        c = lax.axis_index("c")
        s = lax.axis_index("s")

        @pl.loop(0, ZR)
        def _(j):
            zrows[j, pl.ds(0, 16)] = jnp.zeros((16,), jnp.float32)
            zrows[j, pl.ds(16, 16)] = jnp.zeros((16,), jnp.float32)

        for i in range(NE):
            @pl.loop(0, 25)
            def _(z):
                pltpu.sync_copy(zrows, acc.at[pl.ds(s * SEG + z * ZR, ZR)])
            plsc.subcore_barrier()
            pltpu.sync_copy(e01_hbm.at[i, s], idx01)
            hsrc = h_hbm.at[c * NT + SRC_P[i]]

            for b in range(4):
                pltpu.async_copy(hsrc.at[idx0.at[b]], rows4.at[b], gsem.at[b])

            @pl.loop(0, NCHUNK - 5, step=4)
            def _(q, _hsrc=hsrc):
                for b in range(4):
                    pltpu.make_async_copy(
                        _hsrc.at[idx0.at[q + b]], rows4.at[b], gsem.at[b]
                    ).wait()
                    pltpu.async_copy(rows4.at[b], acc.at[idx1.at[q + b]],
                                     ssem.at[b], add=True)
                for b in range(4):
                    pltpu.make_async_copy(rows4.at[b],
                                          acc.at[idx1.at[q + b]],
                                          ssem.at[b]).wait()
                    pltpu.async_copy(_hsrc.at[idx0.at[q + 4 + b]],
                                     rows4.at[b], gsem.at[b])

            for b in range(4):
                qb = NCHUNK - 5 + b
                pltpu.make_async_copy(
                    hsrc.at[idx0.at[qb]], rows4.at[b], gsem.at[b]).wait()
                pltpu.async_copy(rows4.at[b], acc.at[idx1.at[qb]],
                                 ssem.at[b], add=True)
            for b in range(4):
                pltpu.make_async_copy(rows4.at[b],
                                      acc.at[idx1.at[NCHUNK - 5 + b]],
                                      ssem.at[b]).wait()
            pltpu.sync_copy(hsrc.at[idx0.at[NCHUNK - 1]], rows4.at[0])
            pltpu.sync_copy(rows4.at[0], acc.at[idx1.at[NCHUNK - 1]], add=True)

            plsc.subcore_barrier()

            @pl.when(s < NTILE - 1)
            def _(_i=i):
                pltpu.sync_copy(
                    acc.at[pl.ds(s * SEG, SEG)],
                    msg_hbm.at[c * NE + _i, pl.ds(s * SEG, SEG)])

            @pl.when(s == NTILE - 1)
            def _(_i=i):
                last = N - (NTILE - 1) * SEG
                pltpu.sync_copy(
                    acc.at[pl.ds((NTILE - 1) * SEG, last)],
                    msg_hbm.at[c * NE + _i, pl.ds((NTILE - 1) * SEG, last)])

    return k(h2, e01)


def _make_combine_body(k):
    def body(*refs):
        msg_refs = refs[0:k]
        cnt_refs = refs[k:2 * k]
        h_ref = refs[2 * k]
        w_ref = refs[2 * k + 1]
        b_ref = refs[2 * k + 2]
        o_ref = refs[2 * k + 3]
        parts = []
        for j in range(k):
            m = jnp.concatenate([msg_refs[j][0, 0], msg_refs[j][1, 0]], axis=1)
            cnt = cnt_refs[j][0]
            parts.append(m * (1.0 / jnp.maximum(cnt, 1.0)))
        parts.append(jnp.concatenate([h_ref[0, 0], h_ref[1, 0]], axis=1))
        x = jnp.concatenate(parts, axis=1)
        v = (jnp.dot(x, w_ref[...], preferred_element_type=jnp.float32, precision=jax.lax.Precision.HIGHEST)
             + b_ref[...])
        v = jnp.maximum(v, 0.0)
        o_ref[0] = v[:, :HH]
        o_ref[1] = v[:, HH:]
    return body


def _combine_t(t, msg, cnt3, h, wcat, bias):
    grp = GROUPS[t]
    k = len(grp)
    in_specs = []
    args = []
    for j in grp:
        in_specs.append(
            pl.BlockSpec((2, 1, R, HH), lambda r, _j=j: (0, _j, r, 0)))
        args.append(msg)
    for j in grp:
        in_specs.append(pl.BlockSpec((1, R, 1), lambda r, _j=j: (_j, r, 0)))
        args.append(cnt3)
    in_specs.append(pl.BlockSpec((2, 1, R, HH), lambda r: (0, t, r, 0)))
    args.append(h)
    in_specs.append(pl.BlockSpec(((k + 1) * H, H), lambda r: (0, 0)))
    args.append(wcat)
    in_specs.append(pl.BlockSpec((1, H), lambda r: (0, 0)))
    args.append(bias)
    return pl.pallas_call(
        _make_combine_body(k),
        grid=(NRB,),
        in_specs=in_specs,
        out_specs=pl.BlockSpec((2, R, HH), lambda r: (0, r, 0)),
        out_shape=jax.ShapeDtypeStruct((2, N, HH), jnp.float32),
    )(*args)


def _readout_body(h4_ref, h5_ref, w1_ref, b1_ref, w2_ref, b2_ref, o_ref, acc):
    r = pl.program_id(0)

    @pl.when(r == 0)
    def _():
        acc[...] = jnp.zeros_like(acc)

    blk = (jnp.concatenate([h4_ref[0, 0], h4_ref[1, 0]], axis=1)
           + jnp.concatenate([h5_ref[0, 0], h5_ref[1, 0]], axis=1))
    acc[0, :H] += jnp.sum(blk, axis=0)

    @pl.when(r == NRB - 1)
    def _():
        g = jnp.maximum(acc[0, :H] * (1.0 / N), 0.0).reshape(1, H)
        z = jnp.maximum(
            jnp.dot(g, w1_ref[...], preferred_element_type=jnp.float32, precision=jax.lax.Precision.HIGHEST)
            + b1_ref[...], 0.0)
        o_ref[...] = (jnp.dot(z, w2_ref[...], preferred_element_type=jnp.float32, precision=jax.lax.Precision.HIGHEST)
                      + b2_ref[...])


def _readout(h, w1, b1, w2, b2):
    return pl.pallas_call(
        _readout_body,
        grid=(NRB,),
        in_specs=[
            pl.BlockSpec((2, 1, R, HH), lambda r: (0, 4, r, 0)),
            pl.BlockSpec((2, 1, R, HH), lambda r: (0, 5, r, 0)),
            pl.BlockSpec((H, H // 4), lambda r: (0, 0)),
            pl.BlockSpec((1, H // 4), lambda r: (0, 0)),
            pl.BlockSpec((H // 4, 1), lambda r: (0, 0)),
            pl.BlockSpec((1, 1), lambda r: (0, 0)),
        ],
        out_specs=pl.BlockSpec((1, 1), lambda r: (0, 0)),
        out_shape=jax.ShapeDtypeStruct((1, 1), jnp.float32),
        scratch_shapes=[pltpu.VMEM((8, 128), jnp.float32)],
    )(h, h, w1, b1, w2, b2)


def kernel(x_object, W_emb_object, b_emb_object,
           x_ssBox, W_emb_ssBox, b_emb_ssBox,
           x_place_frame, W_emb_place_frame, b_emb_place_frame,
           x_ssCylinder, W_emb_ssCylinder, b_emb_ssCylinder,
           x_pick, W_emb_pick, b_emb_pick,
           x_place, W_emb_place, b_emb_place,
           edge_index, W_l, b_l, W_r, b_r,
           W_out1, b_out1, W_out2, b_out2):
    xs_raw = [x_object, x_ssBox, x_place_frame, x_ssCylinder, x_pick, x_place]
    ws_raw = [W_emb_object, W_emb_ssBox, W_emb_place_frame, W_emb_ssCylinder,
              W_emb_pick, W_emb_place]
    bs_raw = [b_emb_object, b_emb_ssBox, b_emb_place_frame, b_emb_ssCylinder,
              b_emb_pick, b_emb_place]

    xs = jnp.stack([jnp.pad(x, ((0, 0), (0, 8 - d)))
                    for x, d in zip(xs_raw, INDIMS)])
    wemb = jnp.stack([jnp.pad(w, ((0, 8 - d), (0, 0)))
                      for w, d in zip(ws_raw, INDIMS)])
    w3 = wemb.reshape(NT, 8, 2, HH).transpose(2, 0, 1, 3)
    b3 = jnp.stack(bs_raw).reshape(NT, 1, 2, HH).transpose(2, 0, 1, 3)

    order = jnp.array(ORDER, jnp.int32)
    ei = jnp.take(edge_index.astype(jnp.int32), order, axis=0)
    pad0 = jnp.broadcast_to((jnp.arange(E_PAD - E, dtype=jnp.int32) * 97) % N,
                            (NE, E_PAD - E))
    pad1 = jnp.broadcast_to(N + (jnp.arange(E_PAD - E, dtype=jnp.int32) % 8),
                            (NE, E_PAD - E))
    e0 = jnp.concatenate([ei[:, 0, :], pad0], axis=1).reshape(NE, NTILE, NCHUNK, CH)
    e1 = jnp.concatenate([ei[:, 1, :], pad1], axis=1).reshape(NE, NTILE, NCHUNK, CH)
    e01 = jnp.stack([e0, e1], axis=2)

    onehot = np.zeros((NT, NE), np.float32)
    for i_orig, (s_t, d_t) in enumerate(ET):
        onehot[TIDX[d_t], i_orig] = 1.0
    oh = jnp.asarray(onehot)
    wr_comb = jnp.einsum("ti,lihk->lthk", oh, W_r)
    bias_comb = jnp.einsum("ti,lih->lth", oh, b_l + b_r).reshape(L, NT, 1, H)
    wl_p = jnp.take(W_l, order, axis=1)

    cnt = _counts(e1)
    cnt3 = cnt.reshape(NE, N, 1)

    wcats = [[jnp.concatenate([wl_p[l, j] for j in GROUPS[t]]
                              + [wr_comb[l, t]], axis=0)
              for t in range(NT)] for l in range(L)]
    biases = [[bias_comb[l, t] for t in range(NT)] for l in range(L)]

    h = _embed(xs, w3, b3)
    for l in range(L):
        msg = _messages(h.reshape(2 * NT, N, HH), e01)
        msg4 = msg.reshape(2, NE, N, HH)
        h = jnp.stack([_combine_t(t, msg4, cnt3, h, wcats[l][t], biases[l][t])
                       for t in range(NT)], axis=1)
    out = _readout(h, W_out1, b_out1.reshape(1, H // 4),
                   W_out2, b_out2.reshape(1, 1))
    return out.reshape(1)


# speedup vs baseline: 1.1180x; 1.0315x over previous
import functools

import jax
import jax.numpy as jnp
import numpy as np
from jax import lax
from jax.experimental import pallas as pl
from jax.experimental.pallas import tpu as pltpu
from jax.experimental.pallas import tpu_sc as plsc

NODE = ["object", "ssBox", "place_frame", "ssCylinder", "pick", "place"]
INDIMS = [8, 8, 8, 7, 4, 4]
ET = [("object", "ssBox"), ("ssBox", "object"), ("place_frame", "ssBox"),
      ("ssBox", "place_frame"), ("place_frame", "object"), ("object", "place_frame"),
      ("pick", "place"), ("place", "pick"), ("object", "object"),
      ("ssBox", "ssBox"), ("place_frame", "place_frame"), ("ssCylinder", "ssCylinder"),
      ("object", "pick"), ("pick", "object"), ("place_frame", "pick"),
      ("pick", "place_frame"), ("ssCylinder", "pick"), ("pick", "ssCylinder"),
      ("object", "place"), ("place", "object"), ("ssCylinder", "place"),
      ("place", "ssCylinder"), ("place_frame", "place"), ("place", "place_frame")]
TIDX = {t: i for i, t in enumerate(NODE)}
NT = 6
NE = 24
N = 50000
E = 50000
H = 64
L = 3

ORDER = sorted(range(NE), key=lambda i: TIDX[ET[i][1]])
SRC_P = [TIDX[ET[i][0]] for i in ORDER]
DST_P = [TIDX[ET[i][1]] for i in ORDER]
FIRST_MASK = 0
LAST_MASK = 0
for j in range(NE):
    if j == 0 or DST_P[j] != DST_P[j - 1]:
        FIRST_MASK |= 1 << j
    if j == NE - 1 or DST_P[j] != DST_P[j + 1]:
        LAST_MASK |= 1 << j

NTILE = 16
CH = 128
NCHUNK = 25
EPT = CH * NCHUNK
E_PAD = EPT * NTILE
SEG = 3200
ACC_ROWS = SEG * NTILE
ZR = 64
R = 2000
NRB = N // R
GROUPS = [[j for j in range(NE) if DST_P[j] == t] for t in range(NT)]
HH = H // 2


def _embed_body(x_ref, w_ref, b_ref, o_ref):
    o_ref[0, 0] = (
        jnp.dot(x_ref[0], w_ref[0, 0], preferred_element_type=jnp.float32, precision=jax.lax.Precision.HIGHEST)
        + b_ref[0, 0]
    )


def _embed(xs, w3, b3):
    return pl.pallas_call(
        _embed_body,
        grid=(2, NT, NRB),
        in_specs=[
            pl.BlockSpec((1, R, 8), lambda c, t, r: (t, r, 0)),
            pl.BlockSpec((1, 1, 8, HH), lambda c, t, r: (c, t, 0, 0)),
            pl.BlockSpec((1, 1, 1, HH), lambda c, t, r: (c, t, 0, 0)),
        ],
        out_specs=pl.BlockSpec((1, 1, R, HH), lambda c, t, r: (c, t, r, 0)),
        out_shape=jax.ShapeDtypeStruct((2, NT, N, HH), jnp.float32),
    )(xs, w3, b3)


def _counts(e1):
    mesh = plsc.VectorSubcoreMesh(core_axis_name="c", subcore_axis_name="s")
    per_core = NE // 2

    @functools.partial(
        pl.kernel,
        out_type=jax.ShapeDtypeStruct((NE, N), jnp.float32),
        mesh=mesh,
        compiler_params=pltpu.CompilerParams(use_tc_tiling_on_sc=False),
        scratch_types=[
            pltpu.VMEM_SHARED((per_core, ACC_ROWS), jnp.float32),
            pltpu.VMEM((NCHUNK, CH), jnp.int32),
            pltpu.VMEM((CH,), jnp.float32),
            pltpu.VMEM((SEG,), jnp.float32),
        ],
    )
    def k(e1_hbm, cnt_hbm, acc, idxb, ones, zbuf):
        c = lax.axis_index("c")
        s = lax.axis_index("s")

        @pl.loop(0, CH, step=16)
        def _(j):
            ones[pl.ds(j, 16)] = jnp.full((16,), 1.0, jnp.float32)

        @pl.loop(0, SEG, step=16)
        def _(j):
            zbuf[pl.ds(j, 16)] = jnp.zeros((16,), jnp.float32)

        for ii in range(per_core):
            pltpu.sync_copy(zbuf, acc.at[ii, pl.ds(s * SEG, SEG)])
        plsc.subcore_barrier()
        for ii in range(per_core):
            pltpu.sync_copy(e1_hbm.at[c * per_core + ii, s], idxb)

            @pl.loop(0, NCHUNK)
            def _(j, _ii=ii):
                pltpu.sync_copy(ones, acc.at[_ii].at[idxb.at[j]], add=True)

        plsc.subcore_barrier()
        for ii in range(per_core):
            @pl.when(s < NTILE - 1)
            def _(_ii=ii):
                pltpu.sync_copy(
                    acc.at[_ii, pl.ds(s * SEG, SEG)],
                    cnt_hbm.at[c * per_core + _ii, pl.ds(s * SEG, SEG)])

            @pl.when(s == NTILE - 1)
            def _(_ii=ii):
                last = N - (NTILE - 1) * SEG
                pltpu.sync_copy(
                    acc.at[_ii, pl.ds((NTILE - 1) * SEG, last)],
                    cnt_hbm.at[c * per_core + _ii, pl.ds((NTILE - 1) * SEG, last)])

    return k(e1)


def _messages(h2, e01):
    mesh = plsc.VectorSubcoreMesh(core_axis_name="c", subcore_axis_name="s")

    @functools.partial(
        pl.kernel,
        out_type=jax.ShapeDtypeStruct((2 * NE, N, HH), jnp.float32),
        mesh=mesh,
        compiler_params=pltpu.CompilerParams(use_tc_tiling_on_sc=False),
        scratch_types=[
            pltpu.VMEM_SHARED((ACC_ROWS, HH), jnp.float32),
            pltpu.VMEM((2, NCHUNK, CH), jnp.int32),
            pltpu.VMEM((NCHUNK, CH), jnp.int32),
            pltpu.VMEM((4, CH, HH), jnp.float32),
            pltpu.VMEM((ZR, HH), jnp.float32),
            pltpu.SemaphoreType.DMA((4,)),
            pltpu.SemaphoreType.DMA((4,)),
        ],
    )
    def k(h_hbm, e01_hbm, msg_hbm, acc, idx01, idx1, rows4, zrows, gsem, ssem):
        idx0 = idx01.at[0]
        c = lax.axis_index("c")
        s = lax.axis_index("s")

        @pl.loop(0, ZR)
        def _(j):
            zrows[j, pl.ds(0, 16)] = jnp.zeros((16,), jnp.float32)
            zrows[j, pl.ds(16, 16)] = jnp.zeros((16,), jnp.float32)

        for i in range(NE):
            @pl.loop(0, 50)
            def _(z):
                pltpu.sync_copy(zrows, acc.at[pl.ds(s * SEG + z * ZR, ZR)])
            plsc.subcore_barrier()
            pltpu.sync_copy(e01_hbm.at[i, s], idx01)
            pltpu.sync_copy(e01_hbm.at[i, s, 1], idx1)
            hsrc = h_hbm.at[c * NT + SRC_P[i]]

            for b in range(4):
                pltpu.async_copy(hsrc.at[idx0.at[b]], rows4.at[b], gsem.at[b])

            @pl.loop(0, NCHUNK - 5, step=4)
            def _(q, _hsrc=hsrc):
                for b in range(4):
                    pltpu.make_async_copy(
                        _hsrc.at[idx0.at[q + b]], rows4.at[b], gsem.at[b]
                    ).wait()
                    pltpu.sync_copy(rows4.at[b], acc.at[idx1.at[q + b]],
                                    add=True)
                    pltpu.async_copy(_hsrc.at[idx0.at[q + 4 + b]],
                                     rows4.at[b], gsem.at[b])

            for b in range(4):
                qb = NCHUNK - 5 + b
                pltpu.make_async_copy(
                    hsrc.at[idx0.at[qb]], rows4.at[b], gsem.at[b]).wait()
                pltpu.sync_copy(rows4.at[b], acc.at[idx1.at[qb]], add=True)
            pltpu.sync_copy(hsrc.at[idx0.at[NCHUNK - 1]], rows4.at[0])
            pltpu.sync_copy(rows4.at[0], acc.at[idx1.at[NCHUNK - 1]], add=True)

            plsc.subcore_barrier()

            @pl.when(s < NTILE - 1)
            def _(_i=i):
                pltpu.sync_copy(
                    acc.at[pl.ds(s * SEG, SEG)],
                    msg_hbm.at[c * NE + _i, pl.ds(s * SEG, SEG)])

            @pl.when(s == NTILE - 1)
            def _(_i=i):
                last = N - (NTILE - 1) * SEG
                pltpu.sync_copy(
                    acc.at[pl.ds((NTILE - 1) * SEG, last)],
                    msg_hbm.at[c * NE + _i, pl.ds((NTILE - 1) * SEG, last)])

    return k(h2, e01)


def _make_combine_body(k):
    def body(*refs):
        msg_refs = refs[0:k]
        cnt_refs = refs[k:2 * k]
        h_ref = refs[2 * k]
        w_ref = refs[2 * k + 1]
        b_ref = refs[2 * k + 2]
        o_ref = refs[2 * k + 3]
        parts = []
        for j in range(k):
            m = jnp.concatenate([msg_refs[j][0, 0], msg_refs[j][1, 0]], axis=1)
            cnt = cnt_refs[j][0]
            parts.append(m * (1.0 / jnp.maximum(cnt, 1.0)))
        parts.append(jnp.concatenate([h_ref[0, 0], h_ref[1, 0]], axis=1))
        x = jnp.concatenate(parts, axis=1)
        v = (jnp.dot(x, w_ref[...], preferred_element_type=jnp.float32, precision=jax.lax.Precision.HIGHEST)
             + b_ref[...])
        v = jnp.maximum(v, 0.0)
        o_ref[0] = v[:, :HH]
        o_ref[1] = v[:, HH:]
    return body


def _combine_t(t, msg, cnt3, h, wcat, bias):
    grp = GROUPS[t]
    k = len(grp)
    in_specs = []
    args = []
    for j in grp:
        in_specs.append(
            pl.BlockSpec((2, 1, R, HH), lambda r, _j=j: (0, _j, r, 0)))
        args.append(msg)
    for j in grp:
        in_specs.append(pl.BlockSpec((1, R, 1), lambda r, _j=j: (_j, r, 0)))
        args.append(cnt3)
    in_specs.append(pl.BlockSpec((2, 1, R, HH), lambda r: (0, t, r, 0)))
    args.append(h)
    in_specs.append(pl.BlockSpec(((k + 1) * H, H), lambda r: (0, 0)))
    args.append(wcat)
    in_specs.append(pl.BlockSpec((1, H), lambda r: (0, 0)))
    args.append(bias)
    return pl.pallas_call(
        _make_combine_body(k),
        grid=(NRB,),
        in_specs=in_specs,
        out_specs=pl.BlockSpec((2, R, HH), lambda r: (0, r, 0)),
        out_shape=jax.ShapeDtypeStruct((2, N, HH), jnp.float32),
    )(*args)


def _readout_body(h4_ref, h5_ref, w1_ref, b1_ref, w2_ref, b2_ref, o_ref, acc):
    r = pl.program_id(0)

    @pl.when(r == 0)
    def _():
        acc[...] = jnp.zeros_like(acc)

    blk = (jnp.concatenate([h4_ref[0, 0], h4_ref[1, 0]], axis=1)
           + jnp.concatenate([h5_ref[0, 0], h5_ref[1, 0]], axis=1))
    acc[0, :H] += jnp.sum(blk, axis=0)

    @pl.when(r == NRB - 1)
    def _():
        g = jnp.maximum(acc[0, :H] * (1.0 / N), 0.0).reshape(1, H)
        z = jnp.maximum(
            jnp.dot(g, w1_ref[...], preferred_element_type=jnp.float32, precision=jax.lax.Precision.HIGHEST)
            + b1_ref[...], 0.0)
        o_ref[...] = (jnp.dot(z, w2_ref[...], preferred_element_type=jnp.float32, precision=jax.lax.Precision.HIGHEST)
                      + b2_ref[...])


def _readout(h, w1, b1, w2, b2):
    return pl.pallas_call(
        _readout_body,
        grid=(NRB,),
        in_specs=[
            pl.BlockSpec((2, 1, R, HH), lambda r: (0, 4, r, 0)),
            pl.BlockSpec((2, 1, R, HH), lambda r: (0, 5, r, 0)),
            pl.BlockSpec((H, H // 4), lambda r: (0, 0)),
            pl.BlockSpec((1, H // 4), lambda r: (0, 0)),
            pl.BlockSpec((H // 4, 1), lambda r: (0, 0)),
            pl.BlockSpec((1, 1), lambda r: (0, 0)),
        ],
        out_specs=pl.BlockSpec((1, 1), lambda r: (0, 0)),
        out_shape=jax.ShapeDtypeStruct((1, 1), jnp.float32),
        scratch_shapes=[pltpu.VMEM((8, 128), jnp.float32)],
    )(h, h, w1, b1, w2, b2)


def kernel(x_object, W_emb_object, b_emb_object,
           x_ssBox, W_emb_ssBox, b_emb_ssBox,
           x_place_frame, W_emb_place_frame, b_emb_place_frame,
           x_ssCylinder, W_emb_ssCylinder, b_emb_ssCylinder,
           x_pick, W_emb_pick, b_emb_pick,
           x_place, W_emb_place, b_emb_place,
           edge_index, W_l, b_l, W_r, b_r,
           W_out1, b_out1, W_out2, b_out2):
    xs_raw = [x_object, x_ssBox, x_place_frame, x_ssCylinder, x_pick, x_place]
    ws_raw = [W_emb_object, W_emb_ssBox, W_emb_place_frame, W_emb_ssCylinder,
              W_emb_pick, W_emb_place]
    bs_raw = [b_emb_object, b_emb_ssBox, b_emb_place_frame, b_emb_ssCylinder,
              b_emb_pick, b_emb_place]

    xs = jnp.stack([jnp.pad(x, ((0, 0), (0, 8 - d)))
                    for x, d in zip(xs_raw, INDIMS)])
    wemb = jnp.stack([jnp.pad(w, ((0, 8 - d), (0, 0)))
                      for w, d in zip(ws_raw, INDIMS)])
    w3 = wemb.reshape(NT, 8, 2, HH).transpose(2, 0, 1, 3)
    b3 = jnp.stack(bs_raw).reshape(NT, 1, 2, HH).transpose(2, 0, 1, 3)

    order = jnp.array(ORDER, jnp.int32)
    ei = jnp.take(edge_index.astype(jnp.int32), order, axis=0)
    pad0 = jnp.broadcast_to((jnp.arange(E_PAD - E, dtype=jnp.int32) * 97) % N,
                            (NE, E_PAD - E))
    pad1 = jnp.broadcast_to(N + (jnp.arange(E_PAD - E, dtype=jnp.int32) % 8),
                            (NE, E_PAD - E))
    e0 = jnp.concatenate([ei[:, 0, :], pad0], axis=1).reshape(NE, NTILE, NCHUNK, CH)
    e1 = jnp.concatenate([ei[:, 1, :], pad1], axis=1).reshape(NE, NTILE, NCHUNK, CH)
    e01 = jnp.stack([e0, e1], axis=2)

    onehot = np.zeros((NT, NE), np.float32)
    for i_orig, (s_t, d_t) in enumerate(ET):
        onehot[TIDX[d_t], i_orig] = 1.0
    oh = jnp.asarray(onehot)
    wr_comb = jnp.einsum("ti,lihk->lthk", oh, W_r)
    bias_comb = jnp.einsum("ti,lih->lth", oh, b_l + b_r).reshape(L, NT, 1, H)
    wl_p = jnp.take(W_l, order, axis=1)

    cnt = _counts(e1)
    cnt3 = cnt.reshape(NE, N, 1)

    wcats = [[jnp.concatenate([wl_p[l, j] for j in GROUPS[t]]
                              + [wr_comb[l, t]], axis=0)
              for t in range(NT)] for l in range(L)]
    biases = [[bias_comb[l, t] for t in range(NT)] for l in range(L)]

    h = _embed(xs, w3, b3)
    for l in range(L):
        msg = _messages(h.reshape(2 * NT, N, HH), e01)
        msg4 = msg.reshape(2, NE, N, HH)
        h = jnp.stack([_combine_t(t, msg4, cnt3, h, wcats[l][t], biases[l][t])
                       for t in range(NT)], axis=1)
    out = _readout(h, W_out1, b_out1.reshape(1, H // 4),
                   W_out2, b_out2.reshape(1, 1))
    return out.reshape(1)
